# Initial kernel scaffold; baseline (speedup 1.0000x reference)
import jax
import jax.numpy as jnp
from jax.experimental import pallas as pl


def kernel(user_ids, item_ids, timestamps, day_of_week, user_table, item_table, day_table, W0, b0, g0, be0, m0, v0, W1, b1, g1, be1, m1, v1, W2, b2, g2, be2, m2, v2, Wf, bf):
    raise NotImplementedError("write your pallas kernel here")


# speedup vs baseline: 2.4569x; 2.4569x over previous
import functools

import jax
import jax.numpy as jnp
from jax import lax
from jax.experimental import pallas as pl
from jax.experimental.pallas import tpu as pltpu
from jax.experimental.pallas import tpu_sc as plsc

B = 16384
ED = 128


_NC = 2
_NS = 16
_NW = _NC * _NS
_BPW = B // _NW
_IDXW = 128
_NCHUNK = _BPW // _IDXW


def _gather_body(ut, it, uid, iid, ue, ie, idx_v, rows_v, sem):
    wid = lax.axis_index("s") * _NC + lax.axis_index("c")
    base = wid * _BPW
    r0 = wid * _NCHUNK
    pltpu.sync_copy(uid.at[pl.ds(r0, _NCHUNK)], idx_v)
    for j in range(_NCHUNK):
        pltpu.async_copy(ut.at[idx_v.at[j]],
                         rows_v.at[pl.ds(j * _IDXW, _IDXW)], sem).wait()
    pltpu.sync_copy(rows_v, ue.at[pl.ds(base, _BPW)])
    pltpu.sync_copy(iid.at[pl.ds(r0, _NCHUNK)], idx_v)
    for j in range(_NCHUNK):
        pltpu.async_copy(it.at[idx_v.at[j]],
                         rows_v.at[pl.ds(j * _IDXW, _IDXW)], sem).wait()
    pltpu.sync_copy(rows_v, ie.at[pl.ds(base, _BPW)])


@functools.cache
def _make_sc_gather():
    return pl.kernel(
        _gather_body,
        out_type=(jax.ShapeDtypeStruct((B, ED), jnp.float32),
                  jax.ShapeDtypeStruct((B, ED), jnp.float32)),
        mesh=plsc.VectorSubcoreMesh(core_axis_name="c", subcore_axis_name="s"),
        scratch_types=[
            pltpu.VMEM((_NCHUNK, _IDXW), jnp.int32),
            pltpu.VMEM((_BPW, ED), jnp.float32),
            pltpu.SemaphoreType.DMA,
        ],
    )


_TB = 512


def _mlp_body(ue, ie, ts, dow, w0u, w0i, wts, day8, w0d,
              b0, g0, be0, m0, v0,
              w1, b1, g1, be1, m1, v1,
              w2, b2, g2, be2, m2, v2,
              wft, bf, out):
    f32 = jnp.float32
    x_u = ue[...]
    x_i = ie[...]
    h = jnp.dot(x_u, w0u[...], preferred_element_type=f32)
    h += jnp.dot(x_i, w0i[...], preferred_element_type=f32)
    h += ts[...] * wts[...]
    day_w = jnp.dot(day8[...], w0d[...], preferred_element_type=f32)
    oh = (dow[...] == lax.broadcasted_iota(jnp.int32, (1, 8), 1)).astype(f32)
    h += jnp.dot(oh, day_w, preferred_element_type=f32)
    h = (h + b0[...] - m0[...]) * (g0[...] * lax.rsqrt(v0[...] + 1e-5)) + be0[...]
    h = jnp.maximum(h, 0.0)

    h = jnp.dot(h, w1[...], preferred_element_type=f32)
    h = (h + b1[...] - m1[...]) * (g1[...] * lax.rsqrt(v1[...] + 1e-5)) + be1[...]
    h = jnp.maximum(h, 0.0)

    h = jnp.dot(h, w2[...], preferred_element_type=f32)
    h = (h + b2[...] - m2[...]) * (g2[...] * lax.rsqrt(v2[...] + 1e-5)) + be2[...]
    h = jnp.maximum(h, 0.0)

    z = jnp.sum(h * wft[...], axis=1, keepdims=True) + bf[...]
    out[...] = 5.0 / (1.0 + jnp.exp(-z))


def _full(shape):
    return pl.BlockSpec(shape, lambda i: (0, 0))


_mlp = pl.pallas_call(
    _mlp_body,
    grid=(B // _TB,),
    in_specs=[
        pl.BlockSpec((_TB, ED), lambda i: (i, 0)),
        pl.BlockSpec((_TB, ED), lambda i: (i, 0)),
        pl.BlockSpec((_TB, 1), lambda i: (i, 0)),
        pl.BlockSpec((_TB, 1), lambda i: (i, 0)),
        _full((ED, 1024)),
        _full((ED, 1024)),
        _full((1, 1024)),
        _full((8, ED)),
        _full((ED, 1024)),
        _full((1, 1024)), _full((1, 1024)), _full((1, 1024)), _full((1, 1024)), _full((1, 1024)),
        _full((1024, 512)),
        _full((1, 512)), _full((1, 512)), _full((1, 512)), _full((1, 512)), _full((1, 512)),
        _full((512, 256)),
        _full((1, 256)), _full((1, 256)), _full((1, 256)), _full((1, 256)), _full((1, 256)),
        _full((1, 256)),
        _full((1, 1)),
    ],
    out_specs=pl.BlockSpec((_TB, 1), lambda i: (i, 0)),
    out_shape=jax.ShapeDtypeStruct((B, 1), jnp.float32),
    compiler_params=pltpu.CompilerParams(
        dimension_semantics=("parallel",),
    ),
)


def kernel(user_ids, item_ids, timestamps, day_of_week,
           user_table, item_table, day_table,
           W0, b0, g0, be0, m0, v0,
           W1, b1, g1, be1, m1, v1,
           W2, b2, g2, be2, m2, v2,
           Wf, bf):
    uid2 = user_ids.astype(jnp.int32).reshape(B // _IDXW, _IDXW)
    iid2 = item_ids.astype(jnp.int32).reshape(B // _IDXW, _IDXW)
    ue, ie = _make_sc_gather()(user_table, item_table, uid2, iid2)

    w0u = W0[:ED]
    w0i = W0[ED:2 * ED]
    wts = W0[2 * ED:2 * ED + 1]
    w0d = jnp.zeros((ED, 1024), jnp.float32).at[:4].set(W0[2 * ED + 1:])
    day8 = jnp.zeros((8, ED), jnp.float32).at[:7, :4].set(day_table)

    out = _mlp(
        ue, ie, timestamps.reshape(B, 1), day_of_week.astype(jnp.int32).reshape(B, 1),
        w0u, w0i, wts, day8, w0d,
        b0.reshape(1, -1), g0.reshape(1, -1), be0.reshape(1, -1), m0.reshape(1, -1), v0.reshape(1, -1),
        W1,
        b1.reshape(1, -1), g1.reshape(1, -1), be1.reshape(1, -1), m1.reshape(1, -1), v1.reshape(1, -1),
        W2,
        b2.reshape(1, -1), g2.reshape(1, -1), be2.reshape(1, -1), m2.reshape(1, -1), v2.reshape(1, -1),
        Wf.reshape(1, -1), bf.reshape(1, 1),
    )
    return out
